# baseline (speedup 1.0000x reference)
import functools

import jax
import jax.numpy as jnp
from jax import lax
from jax.experimental import pallas as pl
from jax.experimental.pallas import tpu as pltpu
from jax.experimental.pallas import tpu_sc as plsc

KCONST = 10.0
NSEG = 1024
TOTAL = 32768
NS = 16
L = 16
CHUNK = TOTAL // NS
NVEC = CHUNK // L
UNROLL = 2
NOUTER = NVEC // UNROLL
SEG_PER_TILE = NSEG // NS
LN2 = 0.6931471805599453


def _sigmoid(z):
    t = jnp.exp(-jnp.abs(z))
    return jnp.where(z >= 0.0, 1.0 / (1.0 + t), t / (1.0 + t))


def _log(x):
    bits = lax.bitcast_convert_type(x, jnp.int32)
    e = jnp.right_shift(bits, 23) - 127
    m = lax.bitcast_convert_type(
        jnp.bitwise_or(jnp.bitwise_and(bits, 0x007FFFFF), 0x3F800000),
        jnp.float32)
    big = m > 1.4142135
    m = jnp.where(big, m * 0.5, m)
    e = jnp.where(big, e + 1, e)
    t = (m - 1.0) / (m + 1.0)
    t2 = t * t
    atanh2 = t * (2.0 + t2 * (0.66666667 + t2 * (0.4 + t2 * 0.28571429)))
    return e.astype(jnp.float32) * LN2 + atanh2


def _sc_body(vt_hbm, idx_hbm, out_hbm,
             chunk_v, idx_v, e0, s0, e1, s1, sh0, sh1, red, outv, sem):
    sid = lax.axis_index("s")
    base = sid * CHUNK

    h0 = pltpu.async_copy(vt_hbm.at[pl.ds(base, CHUNK)],
                          chunk_v.at[pl.ds(0, CHUNK)], sem)
    h1 = pltpu.async_copy(vt_hbm.at[pl.ds(TOTAL + base, CHUNK)],
                          chunk_v.at[pl.ds(CHUNK, CHUNK)], sem)
    h2 = pltpu.async_copy(vt_hbm.at[pl.ds(2 * TOTAL + base, CHUNK)],
                          chunk_v.at[pl.ds(2 * CHUNK, CHUNK)], sem)
    h3 = pltpu.async_copy(idx_hbm.at[pl.ds(base, CHUNK)],
                          idx_v.at[pl.ds(0, CHUNK)], sem)

    idx_v[pl.ds(CHUNK, L)] = jnp.full((L,), NSEG, jnp.int32)
    zf = jnp.zeros((L,), jnp.float32)

    def zero_body(i, _):
        o = i * (4 * L)
        for j in range(4):
            e0[pl.ds(o + j * L, L)] = zf
            s0[pl.ds(o + j * L, L)] = zf
            e1[pl.ds(o + j * L, L)] = zf
            s1[pl.ds(o + j * L, L)] = zf
        return 0

    lax.fori_loop(0, NSEG // (4 * L), zero_body, 0)
    h0.wait()
    h1.wait()
    h2.wait()
    h3.wait()

    last = jnp.full((L,), L - 1, jnp.int32)

    def main_body(i, carry):
        c0, c1 = carry
        ob = i * (UNROLL * L)
        ids_l, nxt_l, pc0_l, pc1_l = [], [], [], []
        for j in range(UNROLL):
            off = ob + j * L
            ids_l.append(idx_v[pl.ds(off, L)])
            nxt_l.append(idx_v[pl.ds(off + 1, L)])
            a0 = chunk_v[pl.ds(off, L)]
            a1 = (chunk_v[pl.ds(CHUNK + off, L)]
                  + chunk_v[pl.ds(2 * CHUNK + off, L)])
            pc0_l.append(plsc.cumsum(a0))
            pc1_l.append(plsc.cumsum(a1))
        p0_l, p1_l = [], []
        for j in range(UNROLL):
            p0_l.append(pc0_l[j] + c0)
            p1_l.append(pc1_l[j] + c1)
            c0 = c0 + pc0_l[j][last]
            c1 = c1 + pc1_l[j][last]
        for j in range(UNROLL):
            endm = ids_l[j] != nxt_l[j]
            plsc.store_scatter(e0, [ids_l[j]], p0_l[j], mask=endm)
            plsc.store_scatter(s0, [nxt_l[j]], p0_l[j], mask=endm)
            plsc.store_scatter(e1, [ids_l[j]], p1_l[j], mask=endm)
            plsc.store_scatter(s1, [nxt_l[j]], p1_l[j], mask=endm)
        return (c0, c1)

    lax.fori_loop(0, NOUTER, main_body,
                  (jnp.zeros((L,), jnp.float32), jnp.zeros((L,), jnp.float32)))

    def sub_body(i, _):
        o = i * (2 * L)
        for j in range(2):
            oo = o + j * L
            e0[pl.ds(oo, L)] = e0[pl.ds(oo, L)] - s0[pl.ds(oo, L)]
            e1[pl.ds(oo, L)] = e1[pl.ds(oo, L)] - s1[pl.ds(oo, L)]
        return 0

    lax.fori_loop(0, NSEG // (2 * L), sub_body, 0)
    g0 = pltpu.async_copy(e0, sh0.at[pl.ds(sid * NSEG, NSEG)], sem)
    g1 = pltpu.async_copy(e1, sh1.at[pl.ds(sid * NSEG, NSEG)], sem)
    g0.wait()
    g1.wait()
    plsc.subcore_barrier()

    seg0 = sid * SEG_PER_TILE

    def red_start(k, _):
        pltpu.async_copy(
            sh0.at[pl.ds(k * NSEG + seg0, SEG_PER_TILE)],
            red.at[pl.ds(k * SEG_PER_TILE, SEG_PER_TILE)], sem)
        pltpu.async_copy(
            sh1.at[pl.ds(k * NSEG + seg0, SEG_PER_TILE)],
            red.at[pl.ds(NSEG + k * SEG_PER_TILE, SEG_PER_TILE)], sem)
        return 0

    lax.fori_loop(0, NS, red_start, 0)

    def red_wait(k, _):
        pltpu.make_async_copy(
            sh0.at[pl.ds(k * NSEG + seg0, SEG_PER_TILE)],
            red.at[pl.ds(k * SEG_PER_TILE, SEG_PER_TILE)], sem).wait()
        pltpu.make_async_copy(
            sh1.at[pl.ds(k * NSEG + seg0, SEG_PER_TILE)],
            red.at[pl.ds(NSEG + k * SEG_PER_TILE, SEG_PER_TILE)], sem).wait()
        return 0

    lax.fori_loop(0, NS, red_wait, 0)

    for j in range(SEG_PER_TILE // L):
        t0 = [red[pl.ds(k * SEG_PER_TILE + j * L, L)] for k in range(NS)]
        t1 = [red[pl.ds(NSEG + k * SEG_PER_TILE + j * L, L)]
              for k in range(NS)]
        while len(t0) > 1:
            t0 = [t0[k] + t0[k + 1] for k in range(0, len(t0), 2)]
            t1 = [t1[k] + t1[k + 1] for k in range(0, len(t1), 2)]
        l1 = _log(_sigmoid(KCONST * (1.0 - t1[0])) + 1e-10)
        l0 = _log(_sigmoid(KCONST * (5.0 - t0[0])) + 1e-10)
        outv[pl.ds(j * L, L)] = l1
        outv[pl.ds(SEG_PER_TILE + j * L, L)] = l0
    pltpu.sync_copy(outv.at[pl.ds(0, SEG_PER_TILE)],
                    out_hbm.at[0, pl.ds(seg0, SEG_PER_TILE)])
    pltpu.sync_copy(outv.at[pl.ds(SEG_PER_TILE, SEG_PER_TILE)],
                    out_hbm.at[1, pl.ds(seg0, SEG_PER_TILE)])


_sc_run = functools.partial(
    pl.kernel,
    out_type=jax.ShapeDtypeStruct((2, NSEG), jnp.float32),
    mesh=plsc.VectorSubcoreMesh(core_axis_name="c", subcore_axis_name="s",
                                num_cores=1),
    compiler_params=pltpu.CompilerParams(needs_layout_passes=False),
    scratch_types=[
        pltpu.VMEM((3 * CHUNK,), jnp.float32),
        pltpu.VMEM((L + CHUNK,), jnp.int32),
        pltpu.VMEM((NSEG,), jnp.float32),
        pltpu.VMEM((NSEG + 8,), jnp.float32),
        pltpu.VMEM((NSEG,), jnp.float32),
        pltpu.VMEM((NSEG + 8,), jnp.float32),
        pltpu.VMEM_SHARED((NS * NSEG,), jnp.float32),
        pltpu.VMEM_SHARED((NS * NSEG,), jnp.float32),
        pltpu.VMEM((2 * NS * SEG_PER_TILE,), jnp.float32),
        pltpu.VMEM((2 * SEG_PER_TILE,), jnp.float32),
        pltpu.SemaphoreType.DMA,
    ],
)(_sc_body)


def kernel(sub_logits, original_indices):
    vt = sub_logits.T.reshape(-1)
    out2 = _sc_run(vt, original_indices)
    return out2.T


# speedup vs baseline: 1.0198x; 1.0198x over previous
import functools

import jax
import jax.numpy as jnp
from jax import lax
from jax.experimental import pallas as pl
from jax.experimental.pallas import tpu as pltpu
from jax.experimental.pallas import tpu_sc as plsc

KCONST = 10.0
NSEG = 1024
TOTAL = 32768
NS = 16
L = 16
CHUNK = TOTAL // NS
NVEC = CHUNK // L
UNROLL = 4
NOUTER = NVEC // UNROLL
SEG_PER_TILE = NSEG // NS
LN2 = 0.6931471805599453


def _sigmoid(z):
    t = jnp.exp(-jnp.abs(z))
    return jnp.where(z >= 0.0, 1.0 / (1.0 + t), t / (1.0 + t))


def _log(x):
    bits = lax.bitcast_convert_type(x, jnp.int32)
    e = jnp.right_shift(bits, 23) - 127
    m = lax.bitcast_convert_type(
        jnp.bitwise_or(jnp.bitwise_and(bits, 0x007FFFFF), 0x3F800000),
        jnp.float32)
    big = m > 1.4142135
    m = jnp.where(big, m * 0.5, m)
    e = jnp.where(big, e + 1, e)
    t = (m - 1.0) / (m + 1.0)
    t2 = t * t
    atanh2 = t * (2.0 + t2 * (0.66666667 + t2 * (0.4 + t2 * 0.28571429)))
    return e.astype(jnp.float32) * LN2 + atanh2


def _sc_body(vt_hbm, idx_hbm, out_hbm,
             chunk_v, idx_v, e0, s0, e1, s1, sh0, sh1, red, outv, sem):
    sid = lax.axis_index("s")
    base = sid * CHUNK

    h0 = pltpu.async_copy(vt_hbm.at[pl.ds(base, CHUNK)],
                          chunk_v.at[pl.ds(0, CHUNK)], sem)
    h1 = pltpu.async_copy(vt_hbm.at[pl.ds(TOTAL + base, CHUNK)],
                          chunk_v.at[pl.ds(CHUNK, CHUNK)], sem)
    h2 = pltpu.async_copy(vt_hbm.at[pl.ds(2 * TOTAL + base, CHUNK)],
                          chunk_v.at[pl.ds(2 * CHUNK, CHUNK)], sem)
    h3 = pltpu.async_copy(idx_hbm.at[pl.ds(base, CHUNK)],
                          idx_v.at[pl.ds(0, CHUNK)], sem)

    idx_v[pl.ds(CHUNK, L)] = jnp.full((L,), NSEG, jnp.int32)
    zf = jnp.zeros((L,), jnp.float32)

    def zero_body(i, _):
        o = i * (4 * L)
        for j in range(4):
            e0[pl.ds(o + j * L, L)] = zf
            s0[pl.ds(o + j * L, L)] = zf
            e1[pl.ds(o + j * L, L)] = zf
            s1[pl.ds(o + j * L, L)] = zf
        return 0

    lax.fori_loop(0, NSEG // (4 * L), zero_body, 0)
    h0.wait()
    h1.wait()
    h2.wait()
    h3.wait()

    last = jnp.full((L,), L - 1, jnp.int32)

    def main_body(i, carry):
        c0, c1 = carry
        ob = i * (UNROLL * L)
        ids_l, nxt_l, pc0_l, pc1_l = [], [], [], []
        for j in range(UNROLL):
            off = ob + j * L
            ids_l.append(idx_v[pl.ds(off, L)])
            nxt_l.append(idx_v[pl.ds(off + 1, L)])
            a0 = chunk_v[pl.ds(off, L)]
            a1 = (chunk_v[pl.ds(CHUNK + off, L)]
                  + chunk_v[pl.ds(2 * CHUNK + off, L)])
            pc0_l.append(plsc.cumsum(a0))
            pc1_l.append(plsc.cumsum(a1))
        p0_l, p1_l = [], []
        for j in range(UNROLL):
            p0_l.append(pc0_l[j] + c0)
            p1_l.append(pc1_l[j] + c1)
            c0 = c0 + pc0_l[j][last]
            c1 = c1 + pc1_l[j][last]
        for j in range(UNROLL):
            endm = ids_l[j] != nxt_l[j]
            plsc.store_scatter(e0, [ids_l[j]], p0_l[j], mask=endm)
            plsc.store_scatter(s0, [nxt_l[j]], p0_l[j], mask=endm)
            plsc.store_scatter(e1, [ids_l[j]], p1_l[j], mask=endm)
            plsc.store_scatter(s1, [nxt_l[j]], p1_l[j], mask=endm)
        return (c0, c1)

    lax.fori_loop(0, NOUTER, main_body,
                  (jnp.zeros((L,), jnp.float32), jnp.zeros((L,), jnp.float32)))

    def sub_body(i, _):
        o = i * (2 * L)
        for j in range(2):
            oo = o + j * L
            e0[pl.ds(oo, L)] = e0[pl.ds(oo, L)] - s0[pl.ds(oo, L)]
            e1[pl.ds(oo, L)] = e1[pl.ds(oo, L)] - s1[pl.ds(oo, L)]
        return 0

    lax.fori_loop(0, NSEG // (2 * L), sub_body, 0)
    g0 = pltpu.async_copy(e0, sh0.at[pl.ds(sid * NSEG, NSEG)], sem)
    g1 = pltpu.async_copy(e1, sh1.at[pl.ds(sid * NSEG, NSEG)], sem)
    g0.wait()
    g1.wait()
    plsc.subcore_barrier()

    seg0 = sid * SEG_PER_TILE

    def red_start(k, _):
        pltpu.async_copy(
            sh0.at[pl.ds(k * NSEG + seg0, SEG_PER_TILE)],
            red.at[pl.ds(k * SEG_PER_TILE, SEG_PER_TILE)], sem)
        pltpu.async_copy(
            sh1.at[pl.ds(k * NSEG + seg0, SEG_PER_TILE)],
            red.at[pl.ds(NSEG + k * SEG_PER_TILE, SEG_PER_TILE)], sem)
        return 0

    lax.fori_loop(0, NS, red_start, 0)

    def red_wait(k, _):
        pltpu.make_async_copy(
            sh0.at[pl.ds(k * NSEG + seg0, SEG_PER_TILE)],
            red.at[pl.ds(k * SEG_PER_TILE, SEG_PER_TILE)], sem).wait()
        pltpu.make_async_copy(
            sh1.at[pl.ds(k * NSEG + seg0, SEG_PER_TILE)],
            red.at[pl.ds(NSEG + k * SEG_PER_TILE, SEG_PER_TILE)], sem).wait()
        return 0

    lax.fori_loop(0, NS, red_wait, 0)

    for j in range(SEG_PER_TILE // L):
        t0 = [red[pl.ds(k * SEG_PER_TILE + j * L, L)] for k in range(NS)]
        t1 = [red[pl.ds(NSEG + k * SEG_PER_TILE + j * L, L)]
              for k in range(NS)]
        while len(t0) > 1:
            t0 = [t0[k] + t0[k + 1] for k in range(0, len(t0), 2)]
            t1 = [t1[k] + t1[k + 1] for k in range(0, len(t1), 2)]
        l1 = _log(_sigmoid(KCONST * (1.0 - t1[0])) + 1e-10)
        l0 = _log(_sigmoid(KCONST * (5.0 - t0[0])) + 1e-10)
        outv[pl.ds(j * L, L)] = l1
        outv[pl.ds(SEG_PER_TILE + j * L, L)] = l0
    pltpu.sync_copy(outv.at[pl.ds(0, SEG_PER_TILE)],
                    out_hbm.at[0, pl.ds(seg0, SEG_PER_TILE)])
    pltpu.sync_copy(outv.at[pl.ds(SEG_PER_TILE, SEG_PER_TILE)],
                    out_hbm.at[1, pl.ds(seg0, SEG_PER_TILE)])


_sc_run = functools.partial(
    pl.kernel,
    out_type=jax.ShapeDtypeStruct((2, NSEG), jnp.float32),
    mesh=plsc.VectorSubcoreMesh(core_axis_name="c", subcore_axis_name="s",
                                num_cores=1),
    compiler_params=pltpu.CompilerParams(needs_layout_passes=False),
    scratch_types=[
        pltpu.VMEM((3 * CHUNK,), jnp.float32),
        pltpu.VMEM((L + CHUNK,), jnp.int32),
        pltpu.VMEM((NSEG,), jnp.float32),
        pltpu.VMEM((NSEG + 8,), jnp.float32),
        pltpu.VMEM((NSEG,), jnp.float32),
        pltpu.VMEM((NSEG + 8,), jnp.float32),
        pltpu.VMEM_SHARED((NS * NSEG,), jnp.float32),
        pltpu.VMEM_SHARED((NS * NSEG,), jnp.float32),
        pltpu.VMEM((2 * NS * SEG_PER_TILE,), jnp.float32),
        pltpu.VMEM((2 * SEG_PER_TILE,), jnp.float32),
        pltpu.SemaphoreType.DMA,
    ],
)(_sc_body)


def kernel(sub_logits, original_indices):
    vt = sub_logits.T.reshape(-1)
    out2 = _sc_run(vt, original_indices)
    return out2.T
